# Initial kernel scaffold; baseline (speedup 1.0000x reference)
import jax
import jax.numpy as jnp
from jax.experimental import pallas as pl


def kernel(actions, enc_W1, enc_b1, enc_W2, enc_b2, enc_W3, enc_b3, dec_W1, dec_b1, dec_W2, dec_b2, dec_W3, dec_b3, codebooks):
    raise NotImplementedError("write your pallas kernel here")


# speedup vs baseline: 1.0920x; 1.0920x over previous
import functools

import jax
import jax.numpy as jnp
from jax.experimental import pallas as pl
from jax.experimental.pallas import tpu as pltpu

_B = 4096
_T = 32
_A = 14
_DIN = _T * _A
_H = 1024
_D = 256
_G = 2
_K = 1024

_TILE = 512
_GRID = _B // _TILE


def _body(x_ref, w1_ref, b1_ref, w2_ref, b2_ref, w3_ref, b3_ref,
          dw1_ref, db1_ref, dw2_ref, db2_ref, dw3_ref, db3_ref,
          cb_ref, cbt_ref,
          pred_ref, tot_ref, l1_ref, cbl_ref, pp_ref,
          counts_ref, accres_ref, accl1_ref):
    pid = pl.program_id(0)

    @pl.when(pid == 0)
    def _init():
        counts_ref[...] = jnp.zeros_like(counts_ref)
        accres_ref[0, 0] = 0.0
        accl1_ref[0, 0] = 0.0

    x = x_ref[...]
    h = jnp.maximum(jnp.dot(x, w1_ref[...]) + b1_ref[...], 0.0)
    h = jnp.maximum(jnp.dot(h, w2_ref[...]) + b2_ref[...], 0.0)
    z = jnp.dot(h, w3_ref[...]) + b3_ref[...]

    r = z
    qs = []
    for g in range(_G):
        cb = cb_ref[g]
        cbt = cbt_ref[g]
        rn = jnp.sum(r * r, axis=1, keepdims=True)
        cn = jnp.sum(cbt * cbt, axis=0, keepdims=True)
        cross = jnp.dot(r, cbt)
        dist = rn - 2.0 * cross + cn
        m = jnp.min(dist, axis=1, keepdims=True)
        iota = jax.lax.broadcasted_iota(jnp.int32, dist.shape, 1)
        idx = jnp.min(jnp.where(dist == m, iota, jnp.int32(_K)),
                      axis=1, keepdims=True)
        onehot = (iota == idx).astype(jnp.float32)
        q = jax.lax.dot(onehot, cb, precision=jax.lax.Precision.HIGHEST)
        counts_ref[g:g + 1, :] += jnp.sum(onehot, axis=0, keepdims=True)
        qs.append(q)
        r = r - q

    qsum = qs[0] + qs[1]
    diff = z - qsum
    accres_ref[0, 0] += jnp.sum(diff * diff)
    zq = z + (qsum - z)

    hd = jnp.maximum(jnp.dot(zq, dw1_ref[...]) + db1_ref[...], 0.0)
    hd = jnp.maximum(jnp.dot(hd, dw2_ref[...]) + db2_ref[...], 0.0)
    pred = jnp.dot(hd, dw3_ref[...]) + db3_ref[...]
    pred_ref[...] = pred
    accl1_ref[0, 0] += jnp.sum(jnp.abs(pred - x))

    @pl.when(pid == _GRID - 1)
    def _fin():
        probs = counts_ref[...] / float(_B)
        ent = jnp.sum(probs * jnp.log(probs + 1e-10), axis=1,
                      keepdims=True)
        ppv = jnp.exp(-ent)
        pp = jnp.sum(ppv) / float(_G)
        m2 = accres_ref[0, 0] / float(_B * _D)
        cbl = m2 + 0.25 * m2
        l1 = accl1_ref[0, 0] / float(_B * _DIN)
        tot = l1 + cbl
        tot_ref[...] = jnp.full((1, 1), tot)
        l1_ref[...] = jnp.full((1, 1), l1)
        cbl_ref[...] = jnp.full((1, 1), cbl)
        pp_ref[...] = jnp.full((1, 1), pp)


@functools.partial(jax.jit, static_argnames=())
def kernel(actions, enc_W1, enc_b1, enc_W2, enc_b2, enc_W3, enc_b3,
           dec_W1, dec_b1, dec_W2, dec_b2, dec_W3, dec_b3, codebooks):
    x = actions.reshape(_B, _DIN)
    cbt = jnp.swapaxes(codebooks, 1, 2)

    full = lambda shape: pl.BlockSpec(shape, lambda i: tuple(0 for _ in shape))
    in_specs = [
        pl.BlockSpec((_TILE, _DIN), lambda i: (i, 0)),
        full((_DIN, _H)), full((1, _H)),
        full((_H, _H)), full((1, _H)),
        full((_H, _D)), full((1, _D)),
        full((_D, _H)), full((1, _H)),
        full((_H, _H)), full((1, _H)),
        full((_H, _DIN)), full((1, _DIN)),
        full((_G, _K, _D)), full((_G, _D, _K)),
    ]
    out_specs = [
        pl.BlockSpec((_TILE, _DIN), lambda i: (i, 0)),
        full((1, 1)), full((1, 1)), full((1, 1)), full((1, 1)),
    ]
    out_shapes = [
        jax.ShapeDtypeStruct((_B, _DIN), jnp.float32),
        jax.ShapeDtypeStruct((1, 1), jnp.float32),
        jax.ShapeDtypeStruct((1, 1), jnp.float32),
        jax.ShapeDtypeStruct((1, 1), jnp.float32),
        jax.ShapeDtypeStruct((1, 1), jnp.float32),
    ]
    pred, tot, l1, cbl, pp = pl.pallas_call(
        _body,
        grid=(_GRID,),
        in_specs=in_specs,
        out_specs=out_specs,
        out_shape=out_shapes,
        scratch_shapes=[
            pltpu.VMEM((_G, _K), jnp.float32),
            pltpu.SMEM((1, 1), jnp.float32),
            pltpu.SMEM((1, 1), jnp.float32),
        ],
        compiler_params=pltpu.CompilerParams(
            vmem_limit_bytes=120 * 1024 * 1024,
        ),
    )(x, enc_W1, enc_b1.reshape(1, _H), enc_W2, enc_b2.reshape(1, _H),
      enc_W3, enc_b3.reshape(1, _D),
      dec_W1, dec_b1.reshape(1, _H), dec_W2, dec_b2.reshape(1, _H),
      dec_W3, dec_b3.reshape(1, _DIN),
      codebooks, cbt)
    return (pred.reshape(_B, _T, _A), tot[0, 0], l1[0, 0], cbl[0, 0],
            pp[0, 0])


# speedup vs baseline: 1.3212x; 1.2098x over previous
import functools

import jax
import jax.numpy as jnp
from jax.experimental import pallas as pl
from jax.experimental.pallas import tpu as pltpu

_B = 4096
_T = 32
_A = 14
_DIN = _T * _A
_H = 1024
_D = 256
_G = 2
_K = 1024

_TILE = 512
_GRID = _B // _TILE


def _body(x_ref, w1_ref, b1_ref, w2_ref, b2_ref, w3_ref, b3_ref,
          dw1_ref, db1_ref, dw2_ref, db2_ref, dw3_ref, db3_ref,
          cbh_ref, cbl_ref2, cbt_ref,
          pred_ref, tot_ref, l1_ref, cbl_ref, pp_ref,
          counts_ref, accres_ref, accl1_ref):
    pid = pl.program_id(0)

    @pl.when(pid == 0)
    def _init():
        counts_ref[...] = jnp.zeros_like(counts_ref)
        accres_ref[0, 0] = 0.0
        accl1_ref[0, 0] = 0.0

    x = x_ref[...]
    h = jnp.maximum(jnp.dot(x, w1_ref[...]) + b1_ref[...], 0.0)
    h = jnp.maximum(jnp.dot(h, w2_ref[...]) + b2_ref[...], 0.0)
    z = jnp.dot(h, w3_ref[...]) + b3_ref[...]

    r = z
    qs = []
    for g in range(_G):
        cbh = cbh_ref[g]
        cbl = cbl_ref2[g]
        cbt = cbt_ref[g]
        rn = jnp.sum(r * r, axis=1, keepdims=True)
        cn = jnp.sum(cbt * cbt, axis=0, keepdims=True)
        cross = jnp.dot(r, cbt)
        dist = rn - 2.0 * cross + cn
        m = jnp.min(dist, axis=1, keepdims=True)
        iota = jax.lax.broadcasted_iota(jnp.int32, dist.shape, 1)
        idx = jnp.min(jnp.where(dist == m, iota, jnp.int32(_K)),
                      axis=1, keepdims=True)
        onehot = (iota == idx).astype(jnp.float32)
        q = jnp.dot(onehot, cbh) + jnp.dot(onehot, cbl)
        counts_ref[g:g + 1, :] += jnp.sum(onehot, axis=0, keepdims=True)
        qs.append(q)
        r = r - q

    qsum = qs[0] + qs[1]
    diff = z - qsum
    accres_ref[0, 0] += jnp.sum(diff * diff)
    zq = z + (qsum - z)

    hd = jnp.maximum(jnp.dot(zq, dw1_ref[...]) + db1_ref[...], 0.0)
    hd = jnp.maximum(jnp.dot(hd, dw2_ref[...]) + db2_ref[...], 0.0)
    pred = jnp.dot(hd, dw3_ref[...]) + db3_ref[...]
    pred_ref[...] = pred
    accl1_ref[0, 0] += jnp.sum(jnp.abs(pred - x))

    @pl.when(pid == _GRID - 1)
    def _fin():
        probs = counts_ref[...] / float(_B)
        ent = jnp.sum(probs * jnp.log(probs + 1e-10), axis=1,
                      keepdims=True)
        ppv = jnp.exp(-ent)
        pp = jnp.sum(ppv) / float(_G)
        m2 = accres_ref[0, 0] / float(_B * _D)
        cbl = m2 + 0.25 * m2
        l1 = accl1_ref[0, 0] / float(_B * _DIN)
        tot = l1 + cbl
        tot_ref[...] = jnp.full((1, 1), tot)
        l1_ref[...] = jnp.full((1, 1), l1)
        cbl_ref[...] = jnp.full((1, 1), cbl)
        pp_ref[...] = jnp.full((1, 1), pp)


@functools.partial(jax.jit, static_argnames=())
def kernel(actions, enc_W1, enc_b1, enc_W2, enc_b2, enc_W3, enc_b3,
           dec_W1, dec_b1, dec_W2, dec_b2, dec_W3, dec_b3, codebooks):
    x = actions.reshape(_B, _DIN)
    cbt = jnp.swapaxes(codebooks, 1, 2)
    cb_hi = codebooks.astype(jnp.bfloat16).astype(jnp.float32)
    cb_lo = codebooks - cb_hi

    full = lambda shape: pl.BlockSpec(shape, lambda i: tuple(0 for _ in shape))
    in_specs = [
        pl.BlockSpec((_TILE, _DIN), lambda i: (i, 0)),
        full((_DIN, _H)), full((1, _H)),
        full((_H, _H)), full((1, _H)),
        full((_H, _D)), full((1, _D)),
        full((_D, _H)), full((1, _H)),
        full((_H, _H)), full((1, _H)),
        full((_H, _DIN)), full((1, _DIN)),
        full((_G, _K, _D)), full((_G, _K, _D)), full((_G, _D, _K)),
    ]
    out_specs = [
        pl.BlockSpec((_TILE, _DIN), lambda i: (i, 0)),
        full((1, 1)), full((1, 1)), full((1, 1)), full((1, 1)),
    ]
    out_shapes = [
        jax.ShapeDtypeStruct((_B, _DIN), jnp.float32),
        jax.ShapeDtypeStruct((1, 1), jnp.float32),
        jax.ShapeDtypeStruct((1, 1), jnp.float32),
        jax.ShapeDtypeStruct((1, 1), jnp.float32),
        jax.ShapeDtypeStruct((1, 1), jnp.float32),
    ]
    pred, tot, l1, cbl, pp = pl.pallas_call(
        _body,
        grid=(_GRID,),
        in_specs=in_specs,
        out_specs=out_specs,
        out_shape=out_shapes,
        scratch_shapes=[
            pltpu.VMEM((_G, _K), jnp.float32),
            pltpu.SMEM((1, 1), jnp.float32),
            pltpu.SMEM((1, 1), jnp.float32),
        ],
        compiler_params=pltpu.CompilerParams(
            vmem_limit_bytes=120 * 1024 * 1024,
        ),
    )(x, enc_W1, enc_b1.reshape(1, _H), enc_W2, enc_b2.reshape(1, _H),
      enc_W3, enc_b3.reshape(1, _D),
      dec_W1, dec_b1.reshape(1, _H), dec_W2, dec_b2.reshape(1, _H),
      dec_W3, dec_b3.reshape(1, _DIN),
      cb_hi, cb_lo, cbt)
    return (pred.reshape(_B, _T, _A), tot[0, 0], l1[0, 0], cbl[0, 0],
            pp[0, 0])
